# Initial kernel scaffold; baseline (speedup 1.0000x reference)
import jax
import jax.numpy as jnp
from jax.experimental import pallas as pl


def kernel(points1, points2, features1, features2, W1, b1, gamma1, beta1, W2, b2, gamma2, beta2):
    raise NotImplementedError("write your pallas kernel here")


# speedup vs baseline: 19.3586x; 19.3586x over previous
import jax
import jax.numpy as jnp
from jax.experimental import pallas as pl

B, N1, N2 = 16, 1024, 4096
C1, C2, O1, O2 = 512, 256, 512, 256
T = 512
NT = N2 // T
CNT = float(B * N2)
BIG = 1e30
EPS = 1e-3


def _stage1_body(p2t_ref, p1_ref, f1_ref, f2_ref, W1_ref, b1_ref,
                 y1_ref, st_ref):
    b = pl.program_id(0)
    t = pl.program_id(1)

    p1 = p1_ref[0]
    d2 = None
    for c in range(3):
        col = p2t_ref[0, :, c:c + 1]
        row = p1[c:c + 1, :]
        diff = col - row
        sq = diff * diff
        d2 = sq if d2 is None else d2 + sq

    lane = jax.lax.broadcasted_iota(jnp.int32, (T, N1), 1)
    dcur = d2
    sels = []
    invs = []
    for _ in range(3):
        m = jnp.min(dcur, axis=1, keepdims=True)
        i = jnp.min(jnp.where(dcur == m, lane, N1), axis=1,
                    keepdims=True)
        sel = lane == i
        dcur = jnp.where(sel, BIG, dcur)
        invs.append(1.0 / jnp.maximum(m, 1e-10))
        sels.append(sel)

    rnorm = 1.0 / (invs[0] + invs[1] + invs[2])
    STw = (jnp.where(sels[0], invs[0] * rnorm, 0.0)
           + jnp.where(sels[1], invs[1] * rnorm, 0.0)
           + jnp.where(sels[2], invs[2] * rnorm, 0.0))

    interp = jax.lax.dot_general(f1_ref[0], STw, (((1,), (1,)), ((), ())),
                                 preferred_element_type=jnp.float32)
    y = (jax.lax.dot_general(W1_ref[:, :C1], interp,
                             (((1,), (0,)), ((), ())),
                             preferred_element_type=jnp.float32)
         + jax.lax.dot_general(W1_ref[:, C1:], f2_ref[0],
                               (((1,), (0,)), ((), ())),
                               preferred_element_type=jnp.float32)
         + b1_ref[...])
    y1_ref[0] = y

    @pl.when(jnp.logical_and(b == 0, t == 0))
    def _():
        st_ref[...] = jnp.zeros_like(st_ref)

    s1 = jnp.sum(y, axis=1, keepdims=True)
    s2 = jnp.sum(y * y, axis=1, keepdims=True)
    st_ref[...] += jnp.concatenate([s1, s2], axis=1)


def _stage2_body(y1_ref, st1_ref, W2_ref, b2_ref, g1_ref, be1_ref,
                 y2_ref, st_ref):
    b = pl.program_id(0)
    t = pl.program_id(1)

    mean = st1_ref[:, 0:1] / CNT
    var = st1_ref[:, 1:2] / CNT - mean * mean
    inv = jax.lax.rsqrt(var + EPS)
    scale = g1_ref[...] * inv
    shift = be1_ref[...] - mean * scale
    x = jnp.maximum(y1_ref[0] * scale + shift, 0.0)
    y = (jax.lax.dot_general(W2_ref[...], x, (((1,), (0,)), ((), ())),
                             preferred_element_type=jnp.float32)
         + b2_ref[...])
    y2_ref[0] = y

    @pl.when(jnp.logical_and(b == 0, t == 0))
    def _():
        st_ref[...] = jnp.zeros_like(st_ref)

    s1 = jnp.sum(y, axis=1, keepdims=True)
    s2 = jnp.sum(y * y, axis=1, keepdims=True)
    st_ref[...] += jnp.concatenate([s1, s2], axis=1)


def _stage3_body(y2_ref, st2_ref, g2_ref, be2_ref, out_ref):
    mean = st2_ref[:, 0:1] / CNT
    var = st2_ref[:, 1:2] / CNT - mean * mean
    inv = jax.lax.rsqrt(var + EPS)
    scale = g2_ref[...] * inv
    shift = be2_ref[...] - mean * scale
    out_ref[0] = jnp.maximum(y2_ref[0] * scale + shift, 0.0)


def kernel(points1, points2, features1, features2, W1, b1, gamma1, beta1,
           W2, b2, gamma2, beta2):
    p2t = jnp.transpose(points2, (0, 2, 1))
    b1c = b1.reshape(O1, 1)
    b2c = b2.reshape(O2, 1)
    g1c = gamma1.reshape(O1, 1)
    be1c = beta1.reshape(O1, 1)
    g2c = gamma2.reshape(O2, 1)
    be2c = beta2.reshape(O2, 1)

    y1, st1 = pl.pallas_call(
        _stage1_body,
        grid=(B, NT),
        in_specs=[
            pl.BlockSpec((1, T, 3), lambda b, t: (b, t, 0)),
            pl.BlockSpec((1, 3, N1), lambda b, t: (b, 0, 0)),
            pl.BlockSpec((1, C1, N1), lambda b, t: (b, 0, 0)),
            pl.BlockSpec((1, C2, T), lambda b, t: (b, 0, t)),
            pl.BlockSpec((O1, C1 + C2), lambda b, t: (0, 0)),
            pl.BlockSpec((O1, 1), lambda b, t: (0, 0)),
        ],
        out_specs=[
            pl.BlockSpec((1, O1, T), lambda b, t: (b, 0, t)),
            pl.BlockSpec((O1, 2), lambda b, t: (0, 0)),
        ],
        out_shape=[
            jax.ShapeDtypeStruct((B, O1, N2), jnp.float32),
            jax.ShapeDtypeStruct((O1, 2), jnp.float32),
        ],
    )(p2t, points1, features1, features2, W1, b1c)

    y2, st2 = pl.pallas_call(
        _stage2_body,
        grid=(B, NT),
        in_specs=[
            pl.BlockSpec((1, O1, T), lambda b, t: (b, 0, t)),
            pl.BlockSpec((O1, 2), lambda b, t: (0, 0)),
            pl.BlockSpec((O2, O1), lambda b, t: (0, 0)),
            pl.BlockSpec((O2, 1), lambda b, t: (0, 0)),
            pl.BlockSpec((O1, 1), lambda b, t: (0, 0)),
            pl.BlockSpec((O1, 1), lambda b, t: (0, 0)),
        ],
        out_specs=[
            pl.BlockSpec((1, O2, T), lambda b, t: (b, 0, t)),
            pl.BlockSpec((O2, 2), lambda b, t: (0, 0)),
        ],
        out_shape=[
            jax.ShapeDtypeStruct((B, O2, N2), jnp.float32),
            jax.ShapeDtypeStruct((O2, 2), jnp.float32),
        ],
    )(y1, st1, W2, b2c, g1c, be1c)

    out = pl.pallas_call(
        _stage3_body,
        grid=(B, NT),
        in_specs=[
            pl.BlockSpec((1, O2, T), lambda b, t: (b, 0, t)),
            pl.BlockSpec((O2, 2), lambda b, t: (0, 0)),
            pl.BlockSpec((O2, 1), lambda b, t: (0, 0)),
            pl.BlockSpec((O2, 1), lambda b, t: (0, 0)),
        ],
        out_specs=pl.BlockSpec((1, O2, T), lambda b, t: (b, 0, t)),
        out_shape=jax.ShapeDtypeStruct((B, O2, N2), jnp.float32),
    )(y2, st2, g2c, be2c)

    return out
